# Initial kernel scaffold; baseline (speedup 1.0000x reference)
import jax
import jax.numpy as jnp
from jax.experimental import pallas as pl


def kernel(x, token_table, scale_table, pos_table):
    raise NotImplementedError("write your pallas kernel here")


# speedup vs baseline: 4.1554x; 4.1554x over previous
import functools

import jax
import jax.numpy as jnp
from jax import lax
from jax.experimental import pallas as pl
from jax.experimental.pallas import tpu as pltpu, tpu_sc as plsc

B = 4096
SEQ_LEN = 256
N_BINS = 256
LATENT_DIM = 64

NUM_CORES = 2
NUM_SUBCORES = 16
NW = NUM_CORES * NUM_SUBCORES
ROWS = B * SEQ_LEN
RPW = ROWS // NW
STEP = 128
STEPS_PER_BLK = 64
BLKS = RPW // (STEP * STEPS_PER_BLK)
CG = LATENT_DIM // 16


def _body(tok_idx, scl_idx, token_tab, scale_tab, pos_tab, out,
          idx_t, idx_s, tbuf, sbuf, obuf, pos_v, sem_t, sem_s):
    wid = lax.axis_index("s") * NUM_CORES + lax.axis_index("c")
    pltpu.sync_copy(pos_tab, pos_v)

    def blk_body(blk, _):
        idx_row0 = wid * (RPW // STEP) + blk * STEPS_PER_BLK
        pltpu.sync_copy(tok_idx.at[pl.ds(idx_row0, STEPS_PER_BLK)], idx_t)
        pltpu.sync_copy(scl_idx.at[pl.ds(idx_row0, STEPS_PER_BLK)], idx_s)

        def step_body(j, _):
            step = blk * STEPS_PER_BLK + j
            row0 = wid * RPW + step * STEP
            par = lax.rem(step, 2)
            cp_t = pltpu.async_copy(token_tab.at[idx_t.at[j]], tbuf, sem_t)
            cp_s = pltpu.async_copy(scale_tab.at[idx_s.at[j]], sbuf, sem_s)
            cp_t.wait()
            cp_s.wait()

            def row_body(r, _):
                for c in range(CG):
                    sl = pl.ds(c * 16, 16)
                    obuf[r, sl] = tbuf[r, sl] + sbuf[r, sl] + pos_v[par, r, sl]
                return 0

            lax.fori_loop(0, STEP, row_body, 0)
            pltpu.sync_copy(obuf, out.at[pl.ds(row0, STEP)])
            return 0

        lax.fori_loop(0, STEPS_PER_BLK, step_body, 0)
        return 0

    lax.fori_loop(0, BLKS, blk_body, 0)


@jax.jit
def _run(tok_idx, scl_idx, token_table, scale_table, pos_table):
    mesh = plsc.VectorSubcoreMesh(core_axis_name="c", subcore_axis_name="s")
    kfn = pl.kernel(
        _body,
        out_type=jax.ShapeDtypeStruct((ROWS, LATENT_DIM), jnp.float32),
        mesh=mesh,
        compiler_params=pltpu.CompilerParams(use_tc_tiling_on_sc=False),
        scratch_types=[
            pltpu.VMEM((STEPS_PER_BLK, STEP), jnp.int32),
            pltpu.VMEM((STEPS_PER_BLK, STEP), jnp.int32),
            pltpu.VMEM((STEP, LATENT_DIM), jnp.float32),
            pltpu.VMEM((STEP, LATENT_DIM), jnp.float32),
            pltpu.VMEM((STEP, LATENT_DIM), jnp.float32),
            pltpu.VMEM((2, STEP, LATENT_DIM), jnp.float32),
            pltpu.SemaphoreType.DMA,
            pltpu.SemaphoreType.DMA,
        ],
    )
    return kfn(tok_idx, scl_idx, token_table, scale_table, pos_table)


def kernel(x, token_table, scale_table, pos_table):
    tok_idx = x[:, 0, :].reshape(ROWS // STEP, STEP)
    scl_idx = x[:, 1, :].reshape(ROWS // STEP, STEP)
    pos3 = pos_table.reshape(2, STEP, LATENT_DIM)
    out = _run(tok_idx, scl_idx, token_table, scale_table, pos3)
    return out.reshape(B, SEQ_LEN, LATENT_DIM)


# speedup vs baseline: 5.2669x; 1.2675x over previous
import jax
import jax.numpy as jnp
from jax import lax
from jax.experimental import pallas as pl
from jax.experimental.pallas import tpu as pltpu, tpu_sc as plsc

B = 4096
SEQ_LEN = 256
N_BINS = 256
LATENT_DIM = 64

NUM_CORES = 2
NUM_SUBCORES = 16
NW = NUM_CORES * NUM_SUBCORES
ROWS = B * SEQ_LEN
RPW = ROWS // NW
STEP = 128
NSTEPS = RPW // STEP
BLK = 16
NBLK = NSTEPS // BLK
CG = LATENT_DIM // 16


def _body(tok_idx, scl_idx, token_tab, scale_tab, pos_tab, out,
          pos_v, obuf, sbuf, idx_t, idx_s,
          sem_gt0, sem_gt1, sem_gs0, sem_gs1, sem_o0, sem_o1, sem_i):
    wid = lax.axis_index("s") * NUM_CORES + lax.axis_index("c")
    idx_base = wid * NSTEPS
    row_base = wid * RPW
    sem_gt = (sem_gt0, sem_gt1)
    sem_gs = (sem_gs0, sem_gs1)
    sem_o = (sem_o0, sem_o1)

    pltpu.sync_copy(pos_tab, pos_v)
    pltpu.sync_copy(tok_idx.at[pl.ds(idx_base, BLK)], idx_t.at[0])
    pltpu.sync_copy(scl_idx.at[pl.ds(idx_base, BLK)], idx_s.at[0])

    def issue_tok(h, ls, m, p):
        pltpu.async_copy(token_tab.at[idx_t.at[h, ls]], obuf.at[m], sem_gt[p])

    def issue_scl(h, ls, p):
        pltpu.async_copy(scale_tab.at[idx_s.at[h, ls]], sbuf.at[p], sem_gs[p])

    def blk_body(blk, _):
        h = lax.rem(blk, 2)

        @pl.when(blk > 0)
        def _wait_idx():
            pltpu.make_async_copy(tok_idx.at[pl.ds(0, BLK)], idx_t.at[h], sem_i).wait()
            pltpu.make_async_copy(scl_idx.at[pl.ds(0, BLK)], idx_s.at[h], sem_i).wait()

        @pl.when(blk + 1 < NBLK)
        def _prefetch_idx():
            nxt = idx_base + (blk + 1) * BLK
            pltpu.async_copy(tok_idx.at[pl.ds(nxt, BLK)], idx_t.at[1 - h], sem_i)
            pltpu.async_copy(scl_idx.at[pl.ds(nxt, BLK)], idx_s.at[1 - h], sem_i)

        for ls0 in (0, 1):
            issue_tok(h, ls0, ls0, ls0)
            issue_scl(h, ls0, ls0)

        def q_body(q, _):
            for m in range(4):
                p = m % 2
                ls = q * 4 + m
                g = blk * BLK + ls
                pltpu.make_async_copy(token_tab.at[pl.ds(0, STEP)],
                                      obuf.at[m], sem_gt[p]).wait()
                pltpu.make_async_copy(scale_tab.at[pl.ds(0, STEP)],
                                      sbuf.at[p], sem_gs[p]).wait()

                @pl.when(g >= 2)
                def _drain_out():
                    pltpu.make_async_copy(token_tab.at[pl.ds(0, STEP)],
                                          obuf.at[(m + 2) % 4], sem_o[p]).wait()

                @pl.when(ls + 2 < BLK)
                def _prefetch_tok():
                    issue_tok(h, ls + 2, (m + 2) % 4, p)

                def row_body(r, _):
                    for c in range(CG):
                        sl = pl.ds(c * 16, 16)
                        v = sbuf[p, r, sl] + pos_v[p, r, sl]
                        plsc.addupdate(obuf.at[m, r, sl], v)
                    return 0

                lax.fori_loop(0, STEP, row_body, 0)

                pltpu.async_copy(obuf.at[m],
                                 out.at[pl.ds(row_base + g * STEP, STEP)],
                                 sem_o[p])

                @pl.when(ls + 2 < BLK)
                def _prefetch_scl():
                    issue_scl(h, ls + 2, p)
            return 0

        lax.fori_loop(0, BLK // 4, q_body, 0)
        return 0

    lax.fori_loop(0, NBLK, blk_body, 0)

    pltpu.make_async_copy(token_tab.at[pl.ds(0, STEP)], obuf.at[0], sem_o0).wait()
    pltpu.make_async_copy(token_tab.at[pl.ds(0, STEP)], obuf.at[1], sem_o1).wait()


@jax.jit
def _run(tok_idx, scl_idx, token_table, scale_table, pos_table):
    mesh = plsc.VectorSubcoreMesh(core_axis_name="c", subcore_axis_name="s")
    kfn = pl.kernel(
        _body,
        out_type=jax.ShapeDtypeStruct((ROWS, LATENT_DIM), jnp.float32),
        mesh=mesh,
        compiler_params=pltpu.CompilerParams(use_tc_tiling_on_sc=False),
        scratch_types=[
            pltpu.VMEM((2, STEP, LATENT_DIM), jnp.float32),
            pltpu.VMEM((4, STEP, LATENT_DIM), jnp.float32),
            pltpu.VMEM((2, STEP, LATENT_DIM), jnp.float32),
            pltpu.VMEM((2, BLK, STEP), jnp.int32),
            pltpu.VMEM((2, BLK, STEP), jnp.int32),
            pltpu.SemaphoreType.DMA,
            pltpu.SemaphoreType.DMA,
            pltpu.SemaphoreType.DMA,
            pltpu.SemaphoreType.DMA,
            pltpu.SemaphoreType.DMA,
            pltpu.SemaphoreType.DMA,
            pltpu.SemaphoreType.DMA,
        ],
    )
    return kfn(tok_idx, scl_idx, token_table, scale_table, pos_table)


def kernel(x, token_table, scale_table, pos_table):
    tok_idx = x[:, 0, :].reshape(ROWS // STEP, STEP)
    scl_idx = x[:, 1, :].reshape(ROWS // STEP, STEP)
    pos2 = pos_table.reshape(2, STEP, LATENT_DIM)
    out = _run(tok_idx, scl_idx, token_table, scale_table, pos2)
    return out.reshape(B, SEQ_LEN, LATENT_DIM)
